# baseline (speedup 1.0000x reference)
import functools

import jax
import jax.numpy as jnp
from jax import lax
from jax.experimental import pallas as pl
from jax.experimental.pallas import tpu as pltpu
from jax.experimental.pallas import tpu_sc as plsc

VOCAB = 1000000
EMBED_DIM = 64
SEQ_LEN = 200
BATCH = 4096

NUM_CORES = 2
NUM_SUBCORES = 16
LANES = 16
NUM_WORKERS = NUM_CORES * NUM_SUBCORES

SEQ_PER_WORKER = BATCH // NUM_WORKERS
CHUNKSEQ = 4
IDXBLK = 8
NUM_CHUNKS = SEQ_PER_WORKER // CHUNKSEQ
G1 = 128
G2 = SEQ_LEN - G1
ROWS_PER_ITER = 4


def _sc_kernel(idx_hbm, table_hbm, pos_hbm, out_hbm, idx_v, rows_v, pos_v,
               gsem):
  wid = lax.axis_index("s") * NUM_CORES + lax.axis_index("c")
  seq_base = wid * SEQ_PER_WORKER

  pltpu.sync_copy(pos_hbm, pos_v)

  def fire_gathers(half):
    copies = []
    for b in range(CHUNKSEQ):
      ib = half * CHUNKSEQ + b
      copies.append(
          pltpu.async_copy(table_hbm.at[idx_v.at[ib, pl.ds(0, G1)]],
                           rows_v.at[b, pl.ds(0, G1)], gsem))
      copies.append(
          pltpu.async_copy(table_hbm.at[idx_v.at[ib, pl.ds(G1, G2)]],
                           rows_v.at[b, pl.ds(G1, G2)], gsem))
    return copies

  def add_pos():
    def seq_body(b, carry):
      def body(r0, c2):
        for rr in range(ROWS_PER_ITER):
          row = r0 * ROWS_PER_ITER + rr
          for cb in range(EMBED_DIM // LANES):
            sl = pl.ds(cb * LANES, LANES)
            rows_v[b, row, sl] = rows_v[b, row, sl] + pos_v[row, sl]
        return c2

      lax.fori_loop(0, SEQ_LEN // ROWS_PER_ITER, body, 0)
      return carry

    lax.fori_loop(0, CHUNKSEQ, seq_body, 0)

  def blk_body(blk, carry):
    b0 = pl.multiple_of(seq_base + blk * IDXBLK, 8)
    pltpu.sync_copy(idx_hbm.at[pl.ds(b0, IDXBLK)], idx_v)
    for half in range(IDXBLK // CHUNKSEQ):
      for d in fire_gathers(half):
        d.wait()
      add_pos()
      pltpu.sync_copy(rows_v,
                      out_hbm.at[pl.ds(b0 + half * CHUNKSEQ, CHUNKSEQ)])
    return carry

  lax.fori_loop(0, SEQ_PER_WORKER // IDXBLK, blk_body, 0)


@jax.jit
def _run(idx, word_table, pos_table):
  mesh = plsc.VectorSubcoreMesh(core_axis_name="c", subcore_axis_name="s")
  f = functools.partial(
      pl.kernel,
      mesh=mesh,
      compiler_params=pltpu.CompilerParams(use_tc_tiling_on_sc=False),
      out_type=jax.ShapeDtypeStruct((BATCH, SEQ_LEN, EMBED_DIM),
                                    jnp.float32),
      scratch_types=[
          pltpu.VMEM((IDXBLK, SEQ_LEN), jnp.int32),
          pltpu.VMEM((CHUNKSEQ, SEQ_LEN, EMBED_DIM), jnp.float32),
          pltpu.VMEM((SEQ_LEN, EMBED_DIM), jnp.float32),
          pltpu.SemaphoreType.DMA,
      ],
  )(_sc_kernel)
  return f(idx, word_table, pos_table)


def kernel(input_idx, word_table, pos_table):
  return _run(input_idx.astype(jnp.int32), word_table, pos_table)


# speedup vs baseline: 1.0264x; 1.0264x over previous
import functools

import jax
import jax.numpy as jnp
from jax import lax
from jax.experimental import pallas as pl
from jax.experimental.pallas import tpu as pltpu
from jax.experimental.pallas import tpu_sc as plsc

VOCAB = 1000000
EMBED_DIM = 64
SEQ_LEN = 200
BATCH = 4096

NUM_CORES = 2
NUM_SUBCORES = 16
LANES = 16
NUM_WORKERS = NUM_CORES * NUM_SUBCORES

BW = BATCH // NUM_WORKERS
SBLK = 8
SCHUNK = 4
NBLK = SEQ_LEN // SBLK
ROWS_PER_ITER = 4


def _sc_kernel(idxT_hbm, table_hbm, pos_hbm, out_hbm, idx_v, rows_v, pos_v,
               gsem, ssem):
  wid = lax.axis_index("s") * NUM_CORES + lax.axis_index("c")
  b0 = pl.multiple_of(wid * BW, BW)

  pltpu.sync_copy(pos_hbm, pos_v)

  def blk_body(blk, carry):
    s_blk = pl.multiple_of(blk * SBLK, SBLK)
    pltpu.sync_copy(idxT_hbm.at[pl.ds(s_blk, SBLK), pl.ds(b0, BW)], idx_v)
    for half in range(SBLK // SCHUNK):
      copies = []
      for j in range(SCHUNK):
        copies.append(
            pltpu.async_copy(table_hbm.at[idx_v.at[half * SCHUNK + j]],
                             rows_v.at[pl.ds(j * BW, BW)], gsem))
      for c in copies:
        c.wait()

      s0 = s_blk + half * SCHUNK
      for s in range(SCHUNK):
        def body(r0, c2):
          for rr in range(ROWS_PER_ITER):
            row = s * BW + r0 * ROWS_PER_ITER + rr
            for cb in range(EMBED_DIM // LANES):
              sl = pl.ds(cb * LANES, LANES)
              rows_v[row, sl] = rows_v[row, sl] + pos_v[s0 + s, sl]
          return c2

        lax.fori_loop(0, BW // ROWS_PER_ITER, body, 0)

      stores = []
      for s in range(SCHUNK):
        stores.append(
            pltpu.async_copy(rows_v.at[pl.ds(s * BW, BW)],
                             out_hbm.at[pl.ds(b0, BW), s0 + s, :], ssem))
      for st in stores:
        st.wait()
    return carry

  lax.fori_loop(0, NBLK, blk_body, 0)


@jax.jit
def _run(idxT, word_table, pos_table):
  mesh = plsc.VectorSubcoreMesh(core_axis_name="c", subcore_axis_name="s")
  f = functools.partial(
      pl.kernel,
      mesh=mesh,
      compiler_params=pltpu.CompilerParams(use_tc_tiling_on_sc=False),
      out_type=jax.ShapeDtypeStruct((BATCH, SEQ_LEN, EMBED_DIM),
                                    jnp.float32),
      scratch_types=[
          pltpu.VMEM((SBLK, BW), jnp.int32),
          pltpu.VMEM((SCHUNK * BW, EMBED_DIM), jnp.float32),
          pltpu.VMEM((SEQ_LEN, EMBED_DIM), jnp.float32),
          pltpu.SemaphoreType.DMA,
          pltpu.SemaphoreType.DMA,
      ],
  )(_sc_kernel)
  return f(idxT, word_table, pos_table)


def kernel(input_idx, word_table, pos_table):
  idxT = input_idx.astype(jnp.int32).T
  return _run(idxT, word_table, pos_table)


# speedup vs baseline: 1.2598x; 1.2274x over previous
import functools

import jax
import jax.numpy as jnp
from jax import lax
from jax.experimental import pallas as pl
from jax.experimental.pallas import tpu as pltpu
from jax.experimental.pallas import tpu_sc as plsc

VOCAB = 1000000
EMBED_DIM = 64
SEQ_LEN = 200
BATCH = 4096

NUM_CORES = 2
NUM_SUBCORES = 16
LANES = 16
NUM_WORKERS = NUM_CORES * NUM_SUBCORES

TOTAL = BATCH * SEQ_LEN
PER_WORKER = TOTAL // NUM_WORKERS
CHUNK = 512
SUB = 128
K = CHUNK // SUB
NUM_CHUNKS = PER_WORKER // CHUNK
POS4 = 4 * SEQ_LEN
ROWS_PER_ITER = 4


def _sc_kernel(idx_hbm, table_hbm, pos_hbm, out_hbm, idx_v, rows_v, pos4_v,
               gsem):
  wid = lax.axis_index("s") * NUM_CORES + lax.axis_index("c")
  base = wid * PER_WORKER

  for rep in range(POS4 // SEQ_LEN):
    pltpu.sync_copy(pos_hbm, pos4_v.at[pl.ds(rep * SEQ_LEN, SEQ_LEN)])

  def add_pos(ci):
    p0 = (ci * CHUNK) % SEQ_LEN

    def body(r0, carry):
      for rr in range(ROWS_PER_ITER):
        row = r0 * ROWS_PER_ITER + rr
        for cb in range(EMBED_DIM // LANES):
          sl = pl.ds(cb * LANES, LANES)
          rows_v[row, sl] = rows_v[row, sl] + pos4_v[p0 + row, sl]
      return carry

    lax.fori_loop(0, CHUNK // ROWS_PER_ITER, body, 0)

  for ci in range(NUM_CHUNKS):
    off = base + ci * CHUNK
    pltpu.sync_copy(idx_hbm.at[pl.ds(off, CHUNK)], idx_v)
    copies = []
    for j in range(K):
      copies.append(
          pltpu.async_copy(table_hbm.at[idx_v.at[pl.ds(j * SUB, SUB)]],
                           rows_v.at[pl.ds(j * SUB, SUB)], gsem))
    for c in copies:
      c.wait()
    add_pos(ci)
    pltpu.sync_copy(rows_v, out_hbm.at[pl.ds(off, CHUNK)])


@jax.jit
def _run(idx_flat, word_table, pos_table):
  mesh = plsc.VectorSubcoreMesh(core_axis_name="c", subcore_axis_name="s")
  f = functools.partial(
      pl.kernel,
      mesh=mesh,
      compiler_params=pltpu.CompilerParams(use_tc_tiling_on_sc=False),
      out_type=jax.ShapeDtypeStruct((TOTAL, EMBED_DIM), jnp.float32),
      scratch_types=[
          pltpu.VMEM((CHUNK,), jnp.int32),
          pltpu.VMEM((CHUNK, EMBED_DIM), jnp.float32),
          pltpu.VMEM((POS4, EMBED_DIM), jnp.float32),
          pltpu.SemaphoreType.DMA,
      ],
  )(_sc_kernel)
  return f(idx_flat, word_table, pos_table)


def kernel(input_idx, word_table, pos_table):
  idx_flat = input_idx.astype(jnp.int32).reshape(-1)
  out = _run(idx_flat, word_table, pos_table)
  return out.reshape(BATCH, SEQ_LEN, EMBED_DIM)


# speedup vs baseline: 1.3903x; 1.1036x over previous
import functools

import jax
import jax.numpy as jnp
from jax import lax
from jax.experimental import pallas as pl
from jax.experimental.pallas import tpu as pltpu
from jax.experimental.pallas import tpu_sc as plsc

VOCAB = 1000000
EMBED_DIM = 64
SEQ_LEN = 200
BATCH = 4096

NUM_CORES = 2
NUM_SUBCORES = 16
LANES = 16
NUM_WORKERS = NUM_CORES * NUM_SUBCORES

TOTAL = BATCH * SEQ_LEN
PER_WORKER = TOTAL // NUM_WORKERS
CHUNK = 1024
SUB = 128
K = CHUNK // SUB
NUM_CHUNKS = PER_WORKER // CHUNK
ROWS_PER_ITER = 4


def _sc_kernel(idx_hbm, table_hbm, pos_hbm, out_hbm, idx_v, rows_v, pos_v,
               gsem):
  wid = lax.axis_index("s") * NUM_CORES + lax.axis_index("c")
  base = wid * PER_WORKER

  pltpu.sync_copy(pos_hbm, pos_v)

  def add_pos(off):
    s_row = off // BATCH
    pv = [pos_v[s_row, pl.ds(cb * LANES, LANES)]
          for cb in range(EMBED_DIM // LANES)]

    def body(r0, carry):
      for rr in range(ROWS_PER_ITER):
        row = r0 * ROWS_PER_ITER + rr
        for cb in range(EMBED_DIM // LANES):
          sl = pl.ds(cb * LANES, LANES)
          rows_v[row, sl] = rows_v[row, sl] + pv[cb]
      return carry

    lax.fori_loop(0, CHUNK // ROWS_PER_ITER, body, 0)

  for ci in range(NUM_CHUNKS):
    off = base + ci * CHUNK
    pltpu.sync_copy(idx_hbm.at[pl.ds(off, CHUNK)], idx_v)
    copies = []
    for j in range(K):
      copies.append(
          pltpu.async_copy(table_hbm.at[idx_v.at[pl.ds(j * SUB, SUB)]],
                           rows_v.at[pl.ds(j * SUB, SUB)], gsem))
    for c in copies:
      c.wait()
    add_pos(off)
    pltpu.sync_copy(rows_v, out_hbm.at[pl.ds(off, CHUNK)])


@jax.jit
def _run(idx_flat, word_table, pos_table):
  mesh = plsc.VectorSubcoreMesh(core_axis_name="c", subcore_axis_name="s")
  f = functools.partial(
      pl.kernel,
      mesh=mesh,
      compiler_params=pltpu.CompilerParams(use_tc_tiling_on_sc=False),
      out_type=jax.ShapeDtypeStruct((TOTAL, EMBED_DIM), jnp.float32),
      scratch_types=[
          pltpu.VMEM((CHUNK,), jnp.int32),
          pltpu.VMEM((CHUNK, EMBED_DIM), jnp.float32),
          pltpu.VMEM((SEQ_LEN, EMBED_DIM), jnp.float32),
          pltpu.SemaphoreType.DMA,
      ],
  )(_sc_kernel)
  return f(idx_flat, word_table, pos_table)


def kernel(input_idx, word_table, pos_table):
  idx_flat = input_idx.astype(jnp.int32).T.reshape(-1)
  out = _run(idx_flat, word_table, pos_table)
  return out.reshape(SEQ_LEN, BATCH, EMBED_DIM).transpose(1, 0, 2)


# speedup vs baseline: 1.3934x; 1.0022x over previous
import functools

import jax
import jax.numpy as jnp
from jax import lax
from jax.experimental import pallas as pl
from jax.experimental.pallas import tpu as pltpu
from jax.experimental.pallas import tpu_sc as plsc

VOCAB = 1000000
EMBED_DIM = 64
SEQ_LEN = 200
BATCH = 4096

NUM_CORES = 2
NUM_SUBCORES = 16
LANES = 16
NUM_WORKERS = NUM_CORES * NUM_SUBCORES

TOTAL = BATCH * SEQ_LEN
PER_WORKER = TOTAL // NUM_WORKERS
CHUNK = 1024
SUB = 128
K = CHUNK // SUB
NUM_CHUNKS = PER_WORKER // CHUNK
ROWS_PER_ITER = 4


def _sc_kernel(idx_hbm, table_hbm, pos_hbm, out_hbm, idx_v, rows_v, pos_v,
               gsem):
  wid = lax.axis_index("s") * NUM_CORES + lax.axis_index("c")
  base = wid * PER_WORKER

  pltpu.sync_copy(pos_hbm, pos_v)

  def add_pos(off):
    s_row = off // BATCH
    pv = [pos_v[s_row, pl.ds(cb * LANES, LANES)]
          for cb in range(EMBED_DIM // LANES)]

    def body(r0, carry):
      for rr in range(ROWS_PER_ITER):
        row = r0 * ROWS_PER_ITER + rr
        for cb in range(EMBED_DIM // LANES):
          sl = pl.ds(cb * LANES, LANES)
          rows_v[row, sl] = rows_v[row, sl] + pv[cb]
      return carry

    lax.fori_loop(0, CHUNK // ROWS_PER_ITER, body, 0)

  for ci in range(NUM_CHUNKS):
    off = base + ci * CHUNK
    pltpu.sync_copy(idx_hbm.at[pl.ds(off, CHUNK)], idx_v)
    copies = []
    for j in range(K):
      copies.append(
          pltpu.async_copy(table_hbm.at[idx_v.at[pl.ds(j * SUB, SUB)]],
                           rows_v.at[pl.ds(j * SUB, SUB)], gsem))
    for c in copies:
      c.wait()
    add_pos(off)
    pltpu.sync_copy(rows_v, out_hbm.at[pl.ds(off, CHUNK)])


@jax.jit
def _run(idx_flat, word_table, pos_table):
  mesh = plsc.VectorSubcoreMesh(core_axis_name="c", subcore_axis_name="s")
  f = functools.partial(
      pl.kernel,
      mesh=mesh,
      compiler_params=pltpu.CompilerParams(use_tc_tiling_on_sc=False),
      out_type=jax.ShapeDtypeStruct((TOTAL, EMBED_DIM), jnp.float32),
      scratch_types=[
          pltpu.VMEM((CHUNK,), jnp.int32),
          pltpu.VMEM((CHUNK, EMBED_DIM), jnp.float32),
          pltpu.VMEM((SEQ_LEN, EMBED_DIM), jnp.float32),
          pltpu.SemaphoreType.DMA,
      ],
  )(_sc_kernel)
  return f(idx_flat, word_table, pos_table)


def kernel(input_idx, word_table, pos_table):
  idxT = lax.optimization_barrier(input_idx.astype(jnp.int32).T)
  idx_flat = idxT.reshape(-1)
  out = _run(idx_flat, word_table, pos_table)
  return out.reshape(SEQ_LEN, BATCH, EMBED_DIM).transpose(1, 0, 2)
